# Initial kernel scaffold; baseline (speedup 1.0000x reference)
import jax
import jax.numpy as jnp
from jax.experimental import pallas as pl


def kernel(x, edge_index, batch, W1_rel, W1_root, b1, W2, b2, W3, b3, Wlin, blin):
    raise NotImplementedError("write your pallas kernel here")


# speedup vs baseline: 7.4209x; 7.4209x over previous
import functools

import jax
import jax.numpy as jnp
from jax import lax
from jax.experimental import pallas as pl
from jax.experimental.pallas import tpu as pltpu
from jax.experimental.pallas import tpu_sc as plsc

NC = 2
NS = 16
L = 16
K = 128


def _make_sc_scatter(n_pad, dh, e_pad, with_deg):
    ch = e_pad // (NS * K)
    rows_per_tile = n_pad // NS

    out_type = [jax.ShapeDtypeStruct((NC * n_pad, dh), jnp.float32)]
    if with_deg:
        out_type.append(jax.ShapeDtypeStruct((NC * n_pad, L), jnp.float32))

    scratch = [
        pltpu.VMEM((K,), jnp.int32),
        pltpu.VMEM((K,), jnp.int32),
        pltpu.VMEM((K, dh), jnp.float32),
        pltpu.VMEM((K, L), jnp.float32),
        pltpu.VMEM((K, L), jnp.float32),
        pltpu.VMEM_SHARED((n_pad, dh), jnp.float32),
        pltpu.VMEM_SHARED((n_pad, L), jnp.float32),
        pltpu.SemaphoreType.DMA,
    ]

    mesh = plsc.VectorSubcoreMesh(core_axis_name="c", subcore_axis_name="s")

    @functools.partial(
        pl.kernel, mesh=mesh, out_type=tuple(out_type),
        scratch_types=scratch,
        compiler_params=pltpu.CompilerParams(use_tc_tiling_on_sc=False))
    def k(table_hbm, src_hbm, dst_hbm, zeros_hbm, zeros16_hbm, ones_hbm,
          *rest):
        if with_deg:
            acc_out, deg_out = rest[0], rest[1]
            rest = rest[2:]
        else:
            acc_out = rest[0]
            rest = rest[1:]
        idx_v, didx_v, rows_v, ones_v, z16_v, acc_sh, deg_sh, sem = rest

        c = lax.axis_index("c")
        s = lax.axis_index("s")
        do_deg = with_deg

        base_r = s * rows_per_tile
        chunks = []
        q = 0
        while q < rows_per_tile:
            w = min(K, rows_per_tile - q)
            chunks.append((q, w))
            q += w
        pltpu.sync_copy(zeros_hbm.at[pl.ds(0, K)], rows_v)
        if do_deg:
            pltpu.sync_copy(ones_hbm, ones_v)
            pltpu.sync_copy(zeros16_hbm.at[pl.ds(0, K)], z16_v)
        for q, w in chunks:
            pltpu.sync_copy(rows_v.at[pl.ds(0, w)],
                            acc_sh.at[pl.ds(base_r + q, w)])
            if do_deg:
                pltpu.sync_copy(z16_v.at[pl.ds(0, w)],
                                deg_sh.at[pl.ds(base_r + q, w)])
        plsc.subcore_barrier()

        def body(i, carry):
            base = (s * ch + i) * K
            pltpu.sync_copy(src_hbm.at[pl.ds(base, K)], idx_v)
            pltpu.sync_copy(dst_hbm.at[pl.ds(base, K)], didx_v)
            for j in range(K // L):
                v = idx_v[pl.ds(j * L, L)]
                idx_v[pl.ds(j * L, L)] = v + v + c
            pltpu.async_copy(table_hbm.at[idx_v], rows_v, sem).wait()
            pltpu.sync_copy(rows_v, acc_sh.at[didx_v], add=True)
            if do_deg:
                @pl.when(c == 0)
                def _():
                    pltpu.sync_copy(ones_v, deg_sh.at[didx_v], add=True)
            return carry

        lax.fori_loop(0, ch, body, 0)
        plsc.subcore_barrier()

        obase = c * n_pad + base_r
        for q, w in chunks:
            pltpu.sync_copy(acc_sh.at[pl.ds(base_r + q, w)],
                            rows_v.at[pl.ds(0, w)])
            pltpu.sync_copy(rows_v.at[pl.ds(0, w)],
                            acc_out.at[pl.ds(obase + q, w)])
            if do_deg:
                pltpu.sync_copy(deg_sh.at[pl.ds(base_r + q, w)],
                                z16_v.at[pl.ds(0, w)])
                pltpu.sync_copy(z16_v.at[pl.ds(0, w)],
                                deg_out.at[pl.ds(obase + q, w)])

    return k


def _tc_stage1(n_pad, d, h, grid_r):
    def body(x_ref, aA_ref, aB_ref, deg_ref, w1r_ref, w1x_ref, b1_ref,
             w2_ref, xs2_ref, dis_ref):
        a = jnp.concatenate([aA_ref[...], aB_ref[...]], axis=1)
        h1 = jnp.dot(a, w1r_ref[...], preferred_element_type=jnp.float32)
        h1 += jnp.dot(x_ref[...], w1x_ref[...],
                      preferred_element_type=jnp.float32)
        h1 = jnp.maximum(h1 + b1_ref[...], 0.0)
        t2 = jnp.dot(h1, w2_ref[...], preferred_element_type=jnp.float32)
        dis = lax.rsqrt(1.0 + deg_ref[...])
        dis_ref[...] = dis
        xs2_ref[...] = dis * t2

    r = n_pad // grid_r
    row = lambda i: (i, 0)
    fix = lambda i: (0, 0)
    return pl.pallas_call(
        body,
        grid=(grid_r,),
        in_specs=[
            pl.BlockSpec((r, d), row),
            pl.BlockSpec((r, d // 2), row),
            pl.BlockSpec((r, d // 2), row),
            pl.BlockSpec((r, 1), row),
            pl.BlockSpec((d, h), fix),
            pl.BlockSpec((d, h), fix),
            pl.BlockSpec((1, h), fix),
            pl.BlockSpec((h, h), fix),
        ],
        out_specs=[pl.BlockSpec((r, h), row), pl.BlockSpec((r, 1), row)],
        out_shape=[jax.ShapeDtypeStruct((n_pad, h), jnp.float32),
                   jax.ShapeDtypeStruct((n_pad, 1), jnp.float32)],
    )


def _tc_stage2(n_pad, h, grid_r):
    def body(aA_ref, aB_ref, xs2_ref, dis_ref, b2_ref, w3_ref, xs3_ref):
        dis = dis_ref[...]
        a = jnp.concatenate([aA_ref[...], aB_ref[...]], axis=1)
        h2 = dis * (a + xs2_ref[...]) + b2_ref[...]
        h2 = jnp.maximum(h2, 0.0)
        t3 = jnp.dot(h2, w3_ref[...], preferred_element_type=jnp.float32)
        xs3_ref[...] = dis * t3

    r = n_pad // grid_r
    row = lambda i: (i, 0)
    fix = lambda i: (0, 0)
    return pl.pallas_call(
        body,
        grid=(grid_r,),
        in_specs=[
            pl.BlockSpec((r, h // 2), row),
            pl.BlockSpec((r, h // 2), row),
            pl.BlockSpec((r, h), row),
            pl.BlockSpec((r, 1), row),
            pl.BlockSpec((1, h), fix),
            pl.BlockSpec((h, h), fix),
        ],
        out_specs=[pl.BlockSpec((r, h), row)],
        out_shape=[jax.ShapeDtypeStruct((n_pad, h), jnp.float32)],
    )


def _tc_stage3(n_pad, h, nb, c, grid_r):
    r = n_pad // grid_r

    def body(aA_ref, aB_ref, xs3_ref, dis_ref, b3_ref, batch_ref,
             wlin_ref, blin_ref, xn_ref, out_ref, sums_ref, cnt_ref):
        i = pl.program_id(0)

        @pl.when(i == 0)
        def _():
            sums_ref[...] = jnp.zeros_like(sums_ref)
            cnt_ref[...] = jnp.zeros_like(cnt_ref)

        dis = dis_ref[...]
        a = jnp.concatenate([aA_ref[...], aB_ref[...]], axis=1)
        h3 = dis * (a + xs3_ref[...]) + b3_ref[...]
        seg = lax.broadcasted_iota(jnp.int32, (r, nb), 1)
        onehot = (batch_ref[...] == seg).astype(jnp.float32)
        dn = (((0,), (0,)), ((), ()))
        sums_ref[...] += lax.dot_general(
            onehot, h3, dn, preferred_element_type=jnp.float32)
        cnt_ref[...] += lax.dot_general(
            onehot, jnp.ones((r, h), jnp.float32), dn,
            preferred_element_type=jnp.float32)

        @pl.when(i == grid_r - 1)
        def _():
            pooled = sums_ref[...] / jnp.maximum(cnt_ref[...], 1.0)
            nrm = jnp.sqrt(jnp.sum(pooled * pooled, axis=1, keepdims=True))
            xn = pooled / jnp.maximum(nrm, 1e-12)
            xn_ref[...] = xn
            wl = wlin_ref[...]
            wn = jnp.sqrt(jnp.sum(wl * wl, axis=1, keepdims=True))
            wl = wl / jnp.maximum(wn, 1e-12)
            out_ref[...] = lax.dot_general(
                xn, wl, (((1,), (1,)), ((), ())),
                preferred_element_type=jnp.float32) + blin_ref[...]

    row = lambda i: (i, 0)
    fix = lambda i: (0, 0)
    return pl.pallas_call(
        body,
        grid=(grid_r,),
        in_specs=[
            pl.BlockSpec((r, h // 2), row),
            pl.BlockSpec((r, h // 2), row),
            pl.BlockSpec((r, h), row),
            pl.BlockSpec((r, 1), row),
            pl.BlockSpec((1, h), fix),
            pl.BlockSpec((r, 1), row),
            pl.BlockSpec((c, h), fix),
            pl.BlockSpec((1, c), fix),
        ],
        out_specs=[pl.BlockSpec((nb, h), fix), pl.BlockSpec((nb, c), fix)],
        out_shape=[jax.ShapeDtypeStruct((nb, h), jnp.float32),
                   jax.ShapeDtypeStruct((nb, c), jnp.float32)],
        scratch_shapes=[pltpu.VMEM((nb, h), jnp.float32),
                        pltpu.VMEM((nb, h), jnp.float32)],
    )


def kernel(x, edge_index, batch, W1_rel, W1_root, b1, W2, b2, W3, b3,
           Wlin, blin):
    n, d = x.shape
    h = W1_rel.shape[0]
    c = Wlin.shape[0]
    nb = 64
    e = edge_index.shape[1]

    n_pad = ((n + NS * 8 - 1) // (NS * 8)) * (NS * 8)
    e_pad = ((e + NC * NS * K - 1) // (NC * NS * K)) * (NC * NS * K)
    grid_r = 8
    assert n_pad % grid_r == 0

    f32 = jnp.float32
    x_p = jnp.concatenate([x, jnp.zeros((n_pad - n, d), f32)], axis=0)
    pad_idx = jnp.full((e_pad - e,), n, jnp.int32)
    src = jnp.concatenate([edge_index[0], pad_idx])
    dst = jnp.concatenate([edge_index[1], pad_idx])
    batch_p = jnp.concatenate([batch, jnp.full((n_pad - n,), nb, jnp.int32)])
    batch2d = batch_p.reshape(n_pad, 1)
    dh = d // 2
    zeros = jnp.zeros((2 * n_pad, dh), f32)
    zeros16 = jnp.zeros((n_pad, L), f32)
    ones = jnp.ones((K, L), f32)

    sc1 = _make_sc_scatter(n_pad, dh, e_pad, with_deg=True)
    sc23 = _make_sc_scatter(n_pad, dh, e_pad, with_deg=False)

    acc1, deg2 = sc1(x_p.reshape(2 * n_pad, dh), src, dst,
                     zeros, zeros16, ones)
    deg = deg2[:n_pad, 0].reshape(n_pad, 1)

    xs2, dis = _tc_stage1(n_pad, d, h, grid_r)(
        x_p, acc1[:n_pad], acc1[n_pad:], deg, W1_rel.T, W1_root.T,
        b1.reshape(1, h), W2.T)

    (acc2,) = sc23(xs2.reshape(2 * n_pad, dh), src, dst,
                   zeros, zeros16, ones)
    (xs3,) = _tc_stage2(n_pad, h, grid_r)(
        acc2[:n_pad], acc2[n_pad:], xs2, dis, b2.reshape(1, h), W3.T)

    (acc3,) = sc23(xs3.reshape(2 * n_pad, dh), src, dst,
                   zeros, zeros16, ones)
    xn, out = _tc_stage3(n_pad, h, nb, c, grid_r)(
        acc3[:n_pad], acc3[n_pad:], xs3, dis, b3.reshape(1, h), batch2d,
        Wlin, blin.reshape(1, c))
    return (xn, out)


# speedup vs baseline: 8.4613x; 1.1402x over previous
import functools

import jax
import jax.numpy as jnp
from jax import lax
from jax.experimental import pallas as pl
from jax.experimental.pallas import tpu as pltpu
from jax.experimental.pallas import tpu_sc as plsc

NC = 2
NS = 16
L = 16
K = 128


def _make_sc_scatter(n_pad, dh, e_pad, with_deg):
    SS = 4
    nit = e_pad // (NS * SS * K)
    erow = e_pad // K
    rows_per_tile = n_pad // NS

    out_type = [jax.ShapeDtypeStruct((NC * n_pad, dh), jnp.float32)]
    if with_deg:
        out_type.append(jax.ShapeDtypeStruct((NC * n_pad, L), jnp.float32))

    scratch = [
        pltpu.VMEM((4 * SS, K), jnp.int32),
        pltpu.VMEM((4 * SS, K), jnp.int32),
        pltpu.VMEM((2 * SS * K, dh), jnp.float32),
        pltpu.VMEM((K, L), jnp.float32),
        pltpu.VMEM((K, L), jnp.float32),
        pltpu.VMEM_SHARED((n_pad, dh), jnp.float32),
        pltpu.VMEM_SHARED((n_pad, L), jnp.float32),
        pltpu.SemaphoreType.DMA,
        pltpu.SemaphoreType.DMA,
        pltpu.SemaphoreType.DMA,
        pltpu.SemaphoreType.DMA,
    ]

    mesh = plsc.VectorSubcoreMesh(core_axis_name="c", subcore_axis_name="s")

    @functools.partial(
        pl.kernel, mesh=mesh, out_type=tuple(out_type),
        scratch_types=scratch,
        compiler_params=pltpu.CompilerParams(use_tc_tiling_on_sc=False))
    def k(table_hbm, src2_hbm, dst2_hbm, zeros_hbm, zeros16_hbm, ones_hbm,
          *rest):
        if with_deg:
            acc_out, deg_out = rest[0], rest[1]
            rest = rest[2:]
        else:
            acc_out = rest[0]
            rest = rest[1:]
        (sidx_v, didx_v, rows_v, ones_v, z16_v, acc_sh, deg_sh,
         sem_i, sem_g, sem_s0, sem_s1) = rest
        sem_s = (sem_s0, sem_s1)

        c = lax.axis_index("c")
        s = lax.axis_index("s")
        do_deg = with_deg

        base_r = s * rows_per_tile
        chunks = []
        q = 0
        while q < rows_per_tile:
            w = min(K, rows_per_tile - q)
            chunks.append((q, w))
            q += w
        pltpu.sync_copy(zeros_hbm.at[pl.ds(0, K)], rows_v.at[pl.ds(0, K)])
        if do_deg:
            pltpu.sync_copy(ones_hbm, ones_v)
            pltpu.sync_copy(zeros16_hbm.at[pl.ds(0, K)], z16_v)
        for q, w in chunks:
            pltpu.sync_copy(rows_v.at[pl.ds(0, w)],
                            acc_sh.at[pl.ds(base_r + q, w)])
            if do_deg:
                pltpu.sync_copy(z16_v.at[pl.ds(0, w)],
                                deg_sh.at[pl.ds(base_r + q, w)])
        plsc.subcore_barrier()

        def idx_rows(i):
            return (s * nit + i) * SS

        def fire_idx(i, slot):
            pltpu.async_copy(
                src2_hbm.at[pl.ds(c * erow + idx_rows(i), SS)],
                sidx_v.at[pl.ds(slot * SS, SS)], sem_i)
            pltpu.async_copy(dst2_hbm.at[pl.ds(idx_rows(i), SS)],
                             didx_v.at[pl.ds(slot * SS, SS)], sem_i)

        def drain_idx(slot):
            pltpu.make_async_copy(
                src2_hbm.at[pl.ds(0, SS)],
                sidx_v.at[pl.ds(slot * SS, SS)], sem_i).wait()
            pltpu.make_async_copy(
                dst2_hbm.at[pl.ds(0, SS)],
                didx_v.at[pl.ds(slot * SS, SS)], sem_i).wait()

        def drain_scatters(p):
            for j in range(SS):
                pltpu.make_async_copy(
                    rows_v.at[pl.ds((p * SS + j) * K, K)],
                    acc_sh.at[pl.ds(0, K)], sem_s[p]).wait()
            if do_deg:
                @pl.when(c == 0)
                def _():
                    for j in range(SS):
                        pltpu.make_async_copy(
                            ones_v, deg_sh.at[pl.ds(0, K)],
                            sem_s[p]).wait()

        fire_idx(0, 0)
        assert nit % 4 == 0

        def body(g, carry):
            for u in range(4):
                i = 4 * g + u
                p = u % 2
                drain_idx(u)
                fire_idx(jnp.minimum(i + 1, nit - 1), (u + 1) % 4)

                if u < 2:
                    @pl.when(g >= 1)
                    def _():
                        drain_scatters(p)
                else:
                    drain_scatters(p)

                for j in range(SS):
                    pltpu.async_copy(table_hbm.at[sidx_v.at[u * SS + j]],
                                     rows_v.at[pl.ds((p * SS + j) * K, K)],
                                     sem_g)
                for j in range(SS):
                    pltpu.make_async_copy(
                        table_hbm.at[sidx_v.at[u * SS + j]],
                        rows_v.at[pl.ds((p * SS + j) * K, K)],
                        sem_g).wait()

                for j in range(SS):
                    pltpu.async_copy(rows_v.at[pl.ds((p * SS + j) * K, K)],
                                     acc_sh.at[didx_v.at[u * SS + j]],
                                     sem_s[p], add=True)
                if do_deg:
                    @pl.when(c == 0)
                    def _():
                        for j in range(SS):
                            pltpu.async_copy(
                                ones_v, deg_sh.at[didx_v.at[u * SS + j]],
                                sem_s[p], add=True)
            return carry

        lax.fori_loop(0, nit // 4, body, 0)

        drain_scatters(0)
        drain_scatters(1)
        drain_idx(0)
        plsc.subcore_barrier()

        obase = c * n_pad + base_r
        for q, w in chunks:
            pltpu.sync_copy(acc_sh.at[pl.ds(base_r + q, w)],
                            rows_v.at[pl.ds(0, w)])
            pltpu.sync_copy(rows_v.at[pl.ds(0, w)],
                            acc_out.at[pl.ds(obase + q, w)])
            if do_deg:
                pltpu.sync_copy(deg_sh.at[pl.ds(base_r + q, w)],
                                z16_v.at[pl.ds(0, w)])
                pltpu.sync_copy(z16_v.at[pl.ds(0, w)],
                                deg_out.at[pl.ds(obase + q, w)])

    return k


def _tc_stage1(n_pad, d, h, grid_r):
    def body(x_ref, aA_ref, aB_ref, deg_ref, w1r_ref, w1x_ref, b1_ref,
             w2_ref, xs2_ref, dis_ref):
        a = jnp.concatenate([aA_ref[...], aB_ref[...]], axis=1)
        h1 = jnp.dot(a, w1r_ref[...], preferred_element_type=jnp.float32)
        h1 += jnp.dot(x_ref[...], w1x_ref[...],
                      preferred_element_type=jnp.float32)
        h1 = jnp.maximum(h1 + b1_ref[...], 0.0)
        t2 = jnp.dot(h1, w2_ref[...], preferred_element_type=jnp.float32)
        dis = lax.rsqrt(1.0 + deg_ref[...])
        dis_ref[...] = dis
        xs2_ref[...] = dis * t2

    r = n_pad // grid_r
    row = lambda i: (i, 0)
    fix = lambda i: (0, 0)
    return pl.pallas_call(
        body,
        grid=(grid_r,),
        in_specs=[
            pl.BlockSpec((r, d), row),
            pl.BlockSpec((r, d // 2), row),
            pl.BlockSpec((r, d // 2), row),
            pl.BlockSpec((r, 1), row),
            pl.BlockSpec((d, h), fix),
            pl.BlockSpec((d, h), fix),
            pl.BlockSpec((1, h), fix),
            pl.BlockSpec((h, h), fix),
        ],
        out_specs=[pl.BlockSpec((r, h), row), pl.BlockSpec((r, 1), row)],
        out_shape=[jax.ShapeDtypeStruct((n_pad, h), jnp.float32),
                   jax.ShapeDtypeStruct((n_pad, 1), jnp.float32)],
    )


def _tc_stage2(n_pad, h, grid_r):
    def body(aA_ref, aB_ref, xs2_ref, dis_ref, b2_ref, w3_ref, xs3_ref):
        dis = dis_ref[...]
        a = jnp.concatenate([aA_ref[...], aB_ref[...]], axis=1)
        h2 = dis * (a + xs2_ref[...]) + b2_ref[...]
        h2 = jnp.maximum(h2, 0.0)
        t3 = jnp.dot(h2, w3_ref[...], preferred_element_type=jnp.float32)
        xs3_ref[...] = dis * t3

    r = n_pad // grid_r
    row = lambda i: (i, 0)
    fix = lambda i: (0, 0)
    return pl.pallas_call(
        body,
        grid=(grid_r,),
        in_specs=[
            pl.BlockSpec((r, h // 2), row),
            pl.BlockSpec((r, h // 2), row),
            pl.BlockSpec((r, h), row),
            pl.BlockSpec((r, 1), row),
            pl.BlockSpec((1, h), fix),
            pl.BlockSpec((h, h), fix),
        ],
        out_specs=[pl.BlockSpec((r, h), row)],
        out_shape=[jax.ShapeDtypeStruct((n_pad, h), jnp.float32)],
    )


def _tc_stage3(n_pad, h, nb, c, grid_r):
    r = n_pad // grid_r

    def body(aA_ref, aB_ref, xs3_ref, dis_ref, b3_ref, batch_ref,
             wlin_ref, blin_ref, xn_ref, out_ref, sums_ref, cnt_ref):
        i = pl.program_id(0)

        @pl.when(i == 0)
        def _():
            sums_ref[...] = jnp.zeros_like(sums_ref)
            cnt_ref[...] = jnp.zeros_like(cnt_ref)

        dis = dis_ref[...]
        a = jnp.concatenate([aA_ref[...], aB_ref[...]], axis=1)
        h3 = dis * (a + xs3_ref[...]) + b3_ref[...]
        seg = lax.broadcasted_iota(jnp.int32, (r, nb), 1)
        onehot = (batch_ref[...] == seg).astype(jnp.float32)
        dn = (((0,), (0,)), ((), ()))
        sums_ref[...] += lax.dot_general(
            onehot, h3, dn, preferred_element_type=jnp.float32)
        cnt_ref[...] += lax.dot_general(
            onehot, jnp.ones((r, h), jnp.float32), dn,
            preferred_element_type=jnp.float32)

        @pl.when(i == grid_r - 1)
        def _():
            pooled = sums_ref[...] / jnp.maximum(cnt_ref[...], 1.0)
            nrm = jnp.sqrt(jnp.sum(pooled * pooled, axis=1, keepdims=True))
            xn = pooled / jnp.maximum(nrm, 1e-12)
            xn_ref[...] = xn
            wl = wlin_ref[...]
            wn = jnp.sqrt(jnp.sum(wl * wl, axis=1, keepdims=True))
            wl = wl / jnp.maximum(wn, 1e-12)
            out_ref[...] = lax.dot_general(
                xn, wl, (((1,), (1,)), ((), ())),
                preferred_element_type=jnp.float32) + blin_ref[...]

    row = lambda i: (i, 0)
    fix = lambda i: (0, 0)
    return pl.pallas_call(
        body,
        grid=(grid_r,),
        in_specs=[
            pl.BlockSpec((r, h // 2), row),
            pl.BlockSpec((r, h // 2), row),
            pl.BlockSpec((r, h), row),
            pl.BlockSpec((r, 1), row),
            pl.BlockSpec((1, h), fix),
            pl.BlockSpec((r, 1), row),
            pl.BlockSpec((c, h), fix),
            pl.BlockSpec((1, c), fix),
        ],
        out_specs=[pl.BlockSpec((nb, h), fix), pl.BlockSpec((nb, c), fix)],
        out_shape=[jax.ShapeDtypeStruct((nb, h), jnp.float32),
                   jax.ShapeDtypeStruct((nb, c), jnp.float32)],
        scratch_shapes=[pltpu.VMEM((nb, h), jnp.float32),
                        pltpu.VMEM((nb, h), jnp.float32)],
    )


def kernel(x, edge_index, batch, W1_rel, W1_root, b1, W2, b2, W3, b3,
           Wlin, blin):
    n, d = x.shape
    h = W1_rel.shape[0]
    c = Wlin.shape[0]
    nb = 64
    e = edge_index.shape[1]

    n_pad = ((n + NS * 8 - 1) // (NS * 8)) * (NS * 8)
    e_grain = NS * 4 * K
    e_pad = ((e + e_grain - 1) // e_grain) * e_grain
    grid_r = 8
    assert n_pad % grid_r == 0

    f32 = jnp.float32
    x_p = jnp.concatenate([x, jnp.zeros((n_pad - n, d), f32)], axis=0)
    pad_idx = jnp.full((e_pad - e,), n, jnp.int32)
    src = jnp.concatenate([edge_index[0], pad_idx])
    dst = jnp.concatenate([edge_index[1], pad_idx])
    src2 = jnp.concatenate([2 * src, 2 * src + 1]).reshape(-1, K)
    dst2 = dst.reshape(-1, K)
    batch_p = jnp.concatenate([batch, jnp.full((n_pad - n,), nb, jnp.int32)])
    batch2d = batch_p.reshape(n_pad, 1)
    dh = d // 2
    zeros = jnp.zeros((2 * n_pad, dh), f32)
    zeros16 = jnp.zeros((n_pad, L), f32)
    ones = jnp.ones((K, L), f32)

    sc1 = _make_sc_scatter(n_pad, dh, e_pad, with_deg=True)
    sc23 = _make_sc_scatter(n_pad, dh, e_pad, with_deg=False)

    acc1, deg2 = sc1(x_p.reshape(2 * n_pad, dh), src2, dst2,
                     zeros, zeros16, ones)
    deg = deg2[:n_pad, 0].reshape(n_pad, 1)

    xs2, dis = _tc_stage1(n_pad, d, h, grid_r)(
        x_p, acc1[:n_pad], acc1[n_pad:], deg, W1_rel.T, W1_root.T,
        b1.reshape(1, h), W2.T)

    (acc2,) = sc23(xs2.reshape(2 * n_pad, dh), src2, dst2,
                   zeros, zeros16, ones)
    (xs3,) = _tc_stage2(n_pad, h, grid_r)(
        acc2[:n_pad], acc2[n_pad:], xs2, dis, b2.reshape(1, h), W3.T)

    (acc3,) = sc23(xs3.reshape(2 * n_pad, dh), src2, dst2,
                   zeros, zeros16, ones)
    xn, out = _tc_stage3(n_pad, h, nb, c, grid_r)(
        acc3[:n_pad], acc3[n_pad:], xs3, dis, b3.reshape(1, h), batch2d,
        Wlin, blin.reshape(1, c))
    return (xn, out)


# speedup vs baseline: 8.4709x; 1.0011x over previous
import functools

import jax
import jax.numpy as jnp
from jax import lax
from jax.experimental import pallas as pl
from jax.experimental.pallas import tpu as pltpu
from jax.experimental.pallas import tpu_sc as plsc

NC = 2
NS = 16
L = 16
K = 128


def _make_sc_scatter(n_pad, dh, e_pad, with_deg):
    SS = 4
    nit = e_pad // (NS * SS * K)
    rows_per_tile = n_pad // NS

    out_type = [jax.ShapeDtypeStruct((NC * n_pad, dh), jnp.float32)]
    if with_deg:
        out_type.append(jax.ShapeDtypeStruct((NC * n_pad, L), jnp.float32))

    SK = SS * K
    scratch = [
        pltpu.VMEM((4 * SK,), jnp.int32),
        pltpu.VMEM((4 * SK,), jnp.int32),
        pltpu.VMEM((2 * SK, dh), jnp.float32),
        pltpu.VMEM((SK, L), jnp.float32),
        pltpu.VMEM((K, L), jnp.float32),
        pltpu.VMEM_SHARED((n_pad, dh), jnp.float32),
        pltpu.VMEM_SHARED((n_pad, L), jnp.float32),
        pltpu.SemaphoreType.DMA,
        pltpu.SemaphoreType.DMA,
        pltpu.SemaphoreType.DMA,
        pltpu.SemaphoreType.DMA,
    ]

    mesh = plsc.VectorSubcoreMesh(core_axis_name="c", subcore_axis_name="s")

    @functools.partial(
        pl.kernel, mesh=mesh, out_type=tuple(out_type),
        scratch_types=scratch,
        compiler_params=pltpu.CompilerParams(use_tc_tiling_on_sc=False))
    def k(table_hbm, src2_hbm, dst2_hbm, zeros_hbm, zeros16_hbm, ones_hbm,
          *rest):
        if with_deg:
            acc_out, deg_out = rest[0], rest[1]
            rest = rest[2:]
        else:
            acc_out = rest[0]
            rest = rest[1:]
        (sidx_v, didx_v, rows_v, ones_v, z16_v, acc_sh, deg_sh,
         sem_i, sem_g, sem_s0, sem_s1) = rest
        sem_s = (sem_s0, sem_s1)

        c = lax.axis_index("c")
        s = lax.axis_index("s")
        do_deg = with_deg

        base_r = s * rows_per_tile
        chunks = []
        q = 0
        while q < rows_per_tile:
            w = min(K, rows_per_tile - q)
            chunks.append((q, w))
            q += w
        pltpu.sync_copy(zeros_hbm.at[pl.ds(0, K)], rows_v.at[pl.ds(0, K)])
        if do_deg:
            pltpu.sync_copy(ones_hbm, ones_v)
            pltpu.sync_copy(zeros16_hbm.at[pl.ds(0, K)], z16_v)
        for q, w in chunks:
            pltpu.sync_copy(rows_v.at[pl.ds(0, w)],
                            acc_sh.at[pl.ds(base_r + q, w)])
            if do_deg:
                pltpu.sync_copy(z16_v.at[pl.ds(0, w)],
                                deg_sh.at[pl.ds(base_r + q, w)])
        plsc.subcore_barrier()

        def fire_idx(i, slot):
            base = (s * nit + i) * SK
            pltpu.async_copy(src2_hbm.at[pl.ds(c * e_pad + base, SK)],
                             sidx_v.at[pl.ds(slot * SK, SK)], sem_i)
            pltpu.async_copy(dst2_hbm.at[pl.ds(base, SK)],
                             didx_v.at[pl.ds(slot * SK, SK)], sem_i)

        def drain_idx(slot):
            pltpu.make_async_copy(
                src2_hbm.at[pl.ds(0, SK)],
                sidx_v.at[pl.ds(slot * SK, SK)], sem_i).wait()
            pltpu.make_async_copy(
                dst2_hbm.at[pl.ds(0, SK)],
                didx_v.at[pl.ds(slot * SK, SK)], sem_i).wait()

        def drain_scatters(p):
            pltpu.make_async_copy(rows_v.at[pl.ds(p * SK, SK)],
                                  acc_sh.at[pl.ds(0, SK)], sem_s[p]).wait()
            if do_deg:
                @pl.when(c == 0)
                def _():
                    pltpu.make_async_copy(
                        ones_v, deg_sh.at[pl.ds(0, SK)], sem_s[p]).wait()

        fire_idx(0, 0)
        assert nit % 4 == 0

        def body(g, carry):
            for u in range(4):
                i = 4 * g + u
                p = u % 2
                drain_idx(u)
                fire_idx(jnp.minimum(i + 1, nit - 1), (u + 1) % 4)

                if u < 2:
                    @pl.when(g >= 1)
                    def _():
                        drain_scatters(p)
                else:
                    drain_scatters(p)

                pltpu.async_copy(
                    table_hbm.at[sidx_v.at[pl.ds(u * SK, SK)]],
                    rows_v.at[pl.ds(p * SK, SK)], sem_g)
                pltpu.make_async_copy(
                    table_hbm.at[sidx_v.at[pl.ds(u * SK, SK)]],
                    rows_v.at[pl.ds(p * SK, SK)], sem_g).wait()

                pltpu.async_copy(rows_v.at[pl.ds(p * SK, SK)],
                                 acc_sh.at[didx_v.at[pl.ds(u * SK, SK)]],
                                 sem_s[p], add=True)
                if do_deg:
                    @pl.when(c == 0)
                    def _():
                        pltpu.async_copy(
                            ones_v,
                            deg_sh.at[didx_v.at[pl.ds(u * SK, SK)]],
                            sem_s[p], add=True)
            return carry

        lax.fori_loop(0, nit // 4, body, 0)

        drain_scatters(0)
        drain_scatters(1)
        drain_idx(0)
        plsc.subcore_barrier()

        obase = c * n_pad + base_r
        for q, w in chunks:
            pltpu.sync_copy(acc_sh.at[pl.ds(base_r + q, w)],
                            rows_v.at[pl.ds(0, w)])
            pltpu.sync_copy(rows_v.at[pl.ds(0, w)],
                            acc_out.at[pl.ds(obase + q, w)])
            if do_deg:
                pltpu.sync_copy(deg_sh.at[pl.ds(base_r + q, w)],
                                z16_v.at[pl.ds(0, w)])
                pltpu.sync_copy(z16_v.at[pl.ds(0, w)],
                                deg_out.at[pl.ds(obase + q, w)])

    return k


def _tc_stage1(n_pad, d, h, grid_r):
    def body(x_ref, aA_ref, aB_ref, deg_ref, w1r_ref, w1x_ref, b1_ref,
             w2_ref, xs2_ref, dis_ref):
        a = jnp.concatenate([aA_ref[...], aB_ref[...]], axis=1)
        h1 = jnp.dot(a, w1r_ref[...], preferred_element_type=jnp.float32)
        h1 += jnp.dot(x_ref[...], w1x_ref[...],
                      preferred_element_type=jnp.float32)
        h1 = jnp.maximum(h1 + b1_ref[...], 0.0)
        t2 = jnp.dot(h1, w2_ref[...], preferred_element_type=jnp.float32)
        dis = lax.rsqrt(1.0 + deg_ref[...])
        dis_ref[...] = dis
        xs2_ref[...] = dis * t2

    r = n_pad // grid_r
    row = lambda i: (i, 0)
    fix = lambda i: (0, 0)
    return pl.pallas_call(
        body,
        grid=(grid_r,),
        in_specs=[
            pl.BlockSpec((r, d), row),
            pl.BlockSpec((r, d // 2), row),
            pl.BlockSpec((r, d // 2), row),
            pl.BlockSpec((r, 1), row),
            pl.BlockSpec((d, h), fix),
            pl.BlockSpec((d, h), fix),
            pl.BlockSpec((1, h), fix),
            pl.BlockSpec((h, h), fix),
        ],
        out_specs=[pl.BlockSpec((r, h), row), pl.BlockSpec((r, 1), row)],
        out_shape=[jax.ShapeDtypeStruct((n_pad, h), jnp.float32),
                   jax.ShapeDtypeStruct((n_pad, 1), jnp.float32)],
    )


def _tc_stage2(n_pad, h, grid_r):
    def body(aA_ref, aB_ref, xs2_ref, dis_ref, b2_ref, w3_ref, xs3_ref):
        dis = dis_ref[...]
        a = jnp.concatenate([aA_ref[...], aB_ref[...]], axis=1)
        h2 = dis * (a + xs2_ref[...]) + b2_ref[...]
        h2 = jnp.maximum(h2, 0.0)
        t3 = jnp.dot(h2, w3_ref[...], preferred_element_type=jnp.float32)
        xs3_ref[...] = dis * t3

    r = n_pad // grid_r
    row = lambda i: (i, 0)
    fix = lambda i: (0, 0)
    return pl.pallas_call(
        body,
        grid=(grid_r,),
        in_specs=[
            pl.BlockSpec((r, h // 2), row),
            pl.BlockSpec((r, h // 2), row),
            pl.BlockSpec((r, h), row),
            pl.BlockSpec((r, 1), row),
            pl.BlockSpec((1, h), fix),
            pl.BlockSpec((h, h), fix),
        ],
        out_specs=[pl.BlockSpec((r, h), row)],
        out_shape=[jax.ShapeDtypeStruct((n_pad, h), jnp.float32)],
    )


def _tc_stage3(n_pad, h, nb, c, grid_r):
    r = n_pad // grid_r

    def body(aA_ref, aB_ref, xs3_ref, dis_ref, b3_ref, batch_ref,
             wlin_ref, blin_ref, xn_ref, out_ref, sums_ref, cnt_ref):
        i = pl.program_id(0)

        @pl.when(i == 0)
        def _():
            sums_ref[...] = jnp.zeros_like(sums_ref)
            cnt_ref[...] = jnp.zeros_like(cnt_ref)

        dis = dis_ref[...]
        a = jnp.concatenate([aA_ref[...], aB_ref[...]], axis=1)
        h3 = dis * (a + xs3_ref[...]) + b3_ref[...]
        seg = lax.broadcasted_iota(jnp.int32, (r, nb), 1)
        onehot = (batch_ref[...] == seg).astype(jnp.float32)
        dn = (((0,), (0,)), ((), ()))
        sums_ref[...] += lax.dot_general(
            onehot, h3, dn, preferred_element_type=jnp.float32)
        cnt_ref[...] += lax.dot_general(
            onehot, jnp.ones((r, h), jnp.float32), dn,
            preferred_element_type=jnp.float32)

        @pl.when(i == grid_r - 1)
        def _():
            pooled = sums_ref[...] / jnp.maximum(cnt_ref[...], 1.0)
            nrm = jnp.sqrt(jnp.sum(pooled * pooled, axis=1, keepdims=True))
            xn = pooled / jnp.maximum(nrm, 1e-12)
            xn_ref[...] = xn
            wl = wlin_ref[...]
            wn = jnp.sqrt(jnp.sum(wl * wl, axis=1, keepdims=True))
            wl = wl / jnp.maximum(wn, 1e-12)
            out_ref[...] = lax.dot_general(
                xn, wl, (((1,), (1,)), ((), ())),
                preferred_element_type=jnp.float32) + blin_ref[...]

    row = lambda i: (i, 0)
    fix = lambda i: (0, 0)
    return pl.pallas_call(
        body,
        grid=(grid_r,),
        in_specs=[
            pl.BlockSpec((r, h // 2), row),
            pl.BlockSpec((r, h // 2), row),
            pl.BlockSpec((r, h), row),
            pl.BlockSpec((r, 1), row),
            pl.BlockSpec((1, h), fix),
            pl.BlockSpec((r, 1), row),
            pl.BlockSpec((c, h), fix),
            pl.BlockSpec((1, c), fix),
        ],
        out_specs=[pl.BlockSpec((nb, h), fix), pl.BlockSpec((nb, c), fix)],
        out_shape=[jax.ShapeDtypeStruct((nb, h), jnp.float32),
                   jax.ShapeDtypeStruct((nb, c), jnp.float32)],
        scratch_shapes=[pltpu.VMEM((nb, h), jnp.float32),
                        pltpu.VMEM((nb, h), jnp.float32)],
    )


def kernel(x, edge_index, batch, W1_rel, W1_root, b1, W2, b2, W3, b3,
           Wlin, blin):
    n, d = x.shape
    h = W1_rel.shape[0]
    c = Wlin.shape[0]
    nb = 64
    e = edge_index.shape[1]

    n_pad = ((n + NS * 8 - 1) // (NS * 8)) * (NS * 8)
    e_grain = NS * 4 * K
    e_pad = ((e + e_grain - 1) // e_grain) * e_grain
    grid_r = 8
    assert n_pad % grid_r == 0

    f32 = jnp.float32
    x_p = jnp.concatenate([x, jnp.zeros((n_pad - n, d), f32)], axis=0)
    pad_idx = jnp.full((e_pad - e,), n, jnp.int32)
    src = jnp.concatenate([edge_index[0], pad_idx])
    dst = jnp.concatenate([edge_index[1], pad_idx])
    src2 = jnp.concatenate([2 * src, 2 * src + 1])
    dst2 = dst
    batch_p = jnp.concatenate([batch, jnp.full((n_pad - n,), nb, jnp.int32)])
    batch2d = batch_p.reshape(n_pad, 1)
    dh = d // 2
    zeros = jnp.zeros((2 * n_pad, dh), f32)
    zeros16 = jnp.zeros((n_pad, L), f32)
    ones = jnp.ones((4 * K, L), f32)

    sc1 = _make_sc_scatter(n_pad, dh, e_pad, with_deg=True)
    sc23 = _make_sc_scatter(n_pad, dh, e_pad, with_deg=False)

    acc1, deg2 = sc1(x_p.reshape(2 * n_pad, dh), src2, dst2,
                     zeros, zeros16, ones)
    deg = deg2[:n_pad, 0].reshape(n_pad, 1)

    xs2, dis = _tc_stage1(n_pad, d, h, grid_r)(
        x_p, acc1[:n_pad], acc1[n_pad:], deg, W1_rel.T, W1_root.T,
        b1.reshape(1, h), W2.T)

    (acc2,) = sc23(xs2.reshape(2 * n_pad, dh), src2, dst2,
                   zeros, zeros16, ones)
    (xs3,) = _tc_stage2(n_pad, h, grid_r)(
        acc2[:n_pad], acc2[n_pad:], xs2, dis, b2.reshape(1, h), W3.T)

    (acc3,) = sc23(xs3.reshape(2 * n_pad, dh), src2, dst2,
                   zeros, zeros16, ones)
    xn, out = _tc_stage3(n_pad, h, nb, c, grid_r)(
        acc3[:n_pad], acc3[n_pad:], xs3, dis, b3.reshape(1, h), batch2d,
        Wlin, blin.reshape(1, c))
    return (xn, out)
